# baseline (speedup 1.0000x reference)
import functools

import jax
import jax.numpy as jnp
from jax import lax
from jax.experimental import pallas as pl
from jax.experimental.pallas import tpu as pltpu
from jax.experimental.pallas import tpu_sc as plsc

N = 10000
E = 320000
D = 128
C = 64
EDGE_DIM = 16

WIDTH = 80
NC, NS, L = 2, 16, 16
NW = NC * NS
ER = E // WIDTH
ER_PT = ER // NW
NR_PT = N // NS
ZR = 25

NBLK = 10
NB = N // NBLK
EBLK = 20
EB = E // EBLK

_NEG = -3e38


def _node_body(x_ref, w_ref, asrc_ref, adst_ref,
               xw_ref, av_ref, bv_ref, ms_ref, md_ref):
    i = pl.program_id(0)
    xw = jnp.dot(x_ref[...], w_ref[...], preferred_element_type=jnp.float32)
    a_s = jnp.sum(xw * asrc_ref[...], axis=1)
    a_d = jnp.sum(xw * adst_ref[...], axis=1)
    ext = (lax.broadcasted_iota(jnp.int32, (NB, WIDTH - C), 1) == 0)
    xw_ref[...] = jnp.concatenate([xw, ext.astype(jnp.float32)], axis=1)
    av_ref[...] = a_s.reshape(1, 1, NB)
    bv_ref[...] = a_d.reshape(1, 1, NB)

    @pl.when(i == 0)
    def _():
        ms_ref[0, 0] = jnp.float32(_NEG)
        md_ref[0, 0] = jnp.float32(_NEG)

    ms_ref[0, 0] = jnp.maximum(ms_ref[0, 0], jnp.max(a_s))
    md_ref[0, 0] = jnp.maximum(md_ref[0, 0], jnp.max(a_d))


_node_call = pl.pallas_call(
    _node_body,
    grid=(NBLK,),
    in_specs=[
        pl.BlockSpec((NB, D), lambda i: (i, 0)),
        pl.BlockSpec((D, C), lambda i: (0, 0)),
        pl.BlockSpec((1, C), lambda i: (0, 0)),
        pl.BlockSpec((1, C), lambda i: (0, 0)),
    ],
    out_specs=[
        pl.BlockSpec((NB, WIDTH), lambda i: (i, 0)),
        pl.BlockSpec((1, 1, NB), lambda i: (i, 0, 0)),
        pl.BlockSpec((1, 1, NB), lambda i: (i, 0, 0)),
        pl.BlockSpec(memory_space=pltpu.SMEM),
        pl.BlockSpec(memory_space=pltpu.SMEM),
    ],
    out_shape=[
        jax.ShapeDtypeStruct((N, WIDTH), jnp.float32),
        jax.ShapeDtypeStruct((NBLK, 1, NB), jnp.float32),
        jax.ShapeDtypeStruct((NBLK, 1, NB), jnp.float32),
        jax.ShapeDtypeStruct((1, 1), jnp.float32),
        jax.ShapeDtypeStruct((1, 1), jnp.float32),
    ],
)


def _edge_body(ea_ref, we_ref, att_ref, ae_ref, me_ref):
    i = pl.program_id(0)
    ve = jnp.sum(we_ref[...] * att_ref[...], axis=1)
    vcol = jnp.concatenate(
        [ve.reshape(EDGE_DIM, 1),
         jnp.zeros((EDGE_DIM, 7), jnp.float32)], axis=1)
    a8 = jnp.dot(ea_ref[...], vcol,
                 preferred_element_type=jnp.float32)
    at = jnp.swapaxes(a8, 0, 1)
    ae_ref[...] = at[0:1].reshape(1, 1, EB)

    @pl.when(i == 0)
    def _():
        me_ref[0, 0] = jnp.float32(_NEG)

    me_ref[0, 0] = jnp.maximum(me_ref[0, 0], jnp.max(at[0:1]))


_edge_call = pl.pallas_call(
    _edge_body,
    grid=(EBLK,),
    in_specs=[
        pl.BlockSpec((EB, EDGE_DIM), lambda i: (i, 0)),
        pl.BlockSpec((EDGE_DIM, C), lambda i: (0, 0)),
        pl.BlockSpec((1, C), lambda i: (0, 0)),
    ],
    out_specs=[
        pl.BlockSpec((1, 1, EB), lambda i: (i, 0, 0)),
        pl.BlockSpec(memory_space=pltpu.SMEM),
    ],
    out_shape=[
        jax.ShapeDtypeStruct((EBLK, 1, EB), jnp.float32),
        jax.ShapeDtypeStruct((1, 1), jnp.float32),
    ],
)


def _sc_body(xw_hbm, src_hbm, dst_hbm, ae_hbm, asrc_hbm, adst_hbm, m_hbm,
             ea_hbm,
             out_hbm,
             acc_sp, vsrc, vdst, vae, vasrc, vadst, vm, gbuf, gbuf2, gbuf3,
             pbuf, zbuf, sem, sem2, sem3, ssem, ssem2, ssem3):
    del ea_hbm
    cid = lax.axis_index("c")
    sid = lax.axis_index("s")
    tid = cid * NS + sid
    nbase = sid * NR_PT
    ebase = tid * ER_PT

    def zrow(r, carry):
        for b in range(WIDTH // L):
            zbuf[r, pl.ds(b * L, L)] = jnp.zeros((L,), jnp.float32)
        return carry

    lax.fori_loop(0, ZR, zrow, 0)

    def zcopy(k, carry):
        pltpu.sync_copy(zbuf, acc_sp.at[pl.ds(nbase + k * ZR, ZR)])
        return carry

    lax.fori_loop(0, NR_PT // ZR, zcopy, 0)

    pltpu.sync_copy(src_hbm.at[pl.ds(ebase, ER_PT)], vsrc)
    pltpu.sync_copy(dst_hbm.at[pl.ds(ebase, ER_PT)], vdst)
    pltpu.sync_copy(ae_hbm.at[pl.ds(ebase, ER_PT)], vae)
    pltpu.sync_copy(asrc_hbm, vasrc)
    pltpu.sync_copy(adst_hbm, vadst)
    pltpu.sync_copy(m_hbm, vm)
    plsc.subcore_barrier()

    mvec = vm[...]

    def compute_p(ci):
        for g in range(WIDTH // L):
            s16 = vsrc[ci, pl.ds(g * L, L)]
            d16 = vdst[ci, pl.ds(g * L, L)]
            a16 = vae[ci, pl.ds(g * L, L)]
            sv = plsc.load_gather(vasrc, [s16])
            dv = plsc.load_gather(vadst, [d16])
            z = sv + dv + a16
            z = jnp.maximum(z, z * jnp.float32(0.2))
            pbuf[pl.ds(g * L, L)] = jnp.exp(z - mvec)

    def scale(ci, gb):
        del ci
        for g in range(WIDTH // L):
            pv = pbuf[pl.ds(g * L, L)]
            for j in range(L):
                e = g * L + j
                pe = pv[j]
                for b in range(WIDTH // L):
                    gb[e, pl.ds(b * L, L)] = gb[e, pl.ds(b * L, L)] * pe

    def wait_gather(gb, s):
        pltpu.make_async_copy(xw_hbm.at[pl.ds(0, WIDTH)], gb, s).wait()

    def wait_scatter(gb, s):
        pltpu.make_async_copy(gb, acc_sp.at[pl.ds(0, WIDTH)], s).wait()

    gbufs = (gbuf, gbuf2, gbuf3)
    gsems = (sem, sem2, sem3)
    ssems = (ssem, ssem2, ssem3)

    def do_chunk(c, k, start_next):
        gb = gbufs[k]
        compute_p(c)
        wait_gather(gb, gsems[k])
        scale(c, gb)
        pltpu.async_copy(gb, acc_sp.at[vdst.at[c]], ssems[k], add=True)
        if start_next:
            kn = (k + 2) % 3
            wait_scatter(gbufs[kn], ssems[kn])
            pltpu.async_copy(xw_hbm.at[vsrc.at[c + 2]], gbufs[kn], gsems[kn])

    pltpu.async_copy(xw_hbm.at[vsrc.at[0]], gbuf, sem)
    pltpu.async_copy(xw_hbm.at[vsrc.at[1]], gbuf2, sem2)

    def tri(c3, carry):
        c = 3 * c3
        for k in range(3):
            do_chunk(c + k, k, True)
        return carry

    do_chunk(0, 0, False)
    pltpu.async_copy(xw_hbm.at[vsrc.at[2]], gbuf3, sem3)

    def tri1(c3, carry):
        c = 3 * c3 + 1
        for k in range(3):
            do_chunk(c + k, (k + 1) % 3, True)
        return carry

    lax.fori_loop(0, (ER_PT - 5) // 3, tri1, 0)
    do_chunk(121, 1, True)
    do_chunk(122, 2, True)
    do_chunk(123, 0, False)
    do_chunk(124, 1, False)
    wait_scatter(gbuf3, ssem3)
    wait_scatter(gbuf, ssem)
    wait_scatter(gbuf2, ssem2)
    plsc.subcore_barrier()

    pltpu.sync_copy(acc_sp.at[pl.ds(nbase, NR_PT)],
                    out_hbm.at[cid, pl.ds(nbase, NR_PT)])


_sc_call = pl.kernel(
    _sc_body,
    out_type=jax.ShapeDtypeStruct((NC, N, WIDTH), jnp.float32),
    mesh=plsc.VectorSubcoreMesh(core_axis_name="c", subcore_axis_name="s"),
    compiler_params=pltpu.CompilerParams(use_tc_tiling_on_sc=False,
                                         needs_layout_passes=False),
    scratch_types=[
        pltpu.VMEM_SHARED((N, WIDTH), jnp.float32),
        pltpu.VMEM((ER_PT, WIDTH), jnp.int32),
        pltpu.VMEM((ER_PT, WIDTH), jnp.int32),
        pltpu.VMEM((ER_PT, WIDTH), jnp.float32),
        pltpu.VMEM((N,), jnp.float32),
        pltpu.VMEM((N,), jnp.float32),
        pltpu.VMEM((L,), jnp.float32),
        pltpu.VMEM((WIDTH, WIDTH), jnp.float32),
        pltpu.VMEM((WIDTH, WIDTH), jnp.float32),
        pltpu.VMEM((WIDTH, WIDTH), jnp.float32),
        pltpu.VMEM((WIDTH,), jnp.float32),
        pltpu.VMEM((ZR, WIDTH), jnp.float32),
        pltpu.SemaphoreType.DMA,
        pltpu.SemaphoreType.DMA,
        pltpu.SemaphoreType.DMA,
        pltpu.SemaphoreType.DMA,
        pltpu.SemaphoreType.DMA,
        pltpu.SemaphoreType.DMA,
    ],
)


def _final_body(acc_ref, bias_ref, out_ref):
    num = acc_ref[0] + acc_ref[1]
    den = num[:, C:C + 1]
    out_ref[...] = num[:, :C] / (den + jnp.float32(1e-16)) + bias_ref[...]


_final_call = pl.pallas_call(
    _final_body,
    grid=(NBLK,),
    in_specs=[
        pl.BlockSpec((NC, NB, WIDTH), lambda i: (0, i, 0)),
        pl.BlockSpec((1, C), lambda i: (0, 0)),
    ],
    out_specs=pl.BlockSpec((NB, C), lambda i: (i, 0)),
    out_shape=jax.ShapeDtypeStruct((N, C), jnp.float32),
)


def kernel(x, edge_index, edge_attr, W, W_e, att_src, att_dst, att_edge, bias):
    src = edge_index[0].astype(jnp.int32)
    dst = edge_index[1].astype(jnp.int32)
    xw_ext, a_src, a_dst, ms, md = _node_call(
        x, W, att_src.reshape(1, C), att_dst.reshape(1, C))
    a_edge, me = _edge_call(edge_attr, W_e, att_edge.reshape(1, C))
    msum = ms[0, 0] + md[0, 0] + me[0, 0]
    m_shift = jnp.maximum(msum, 0.2 * msum)
    m16 = jnp.full((L,), m_shift, jnp.float32)
    acc = _sc_call(xw_ext, src.reshape(ER, WIDTH), dst.reshape(ER, WIDTH),
                   a_edge.reshape(ER, WIDTH), a_src.reshape(N),
                   a_dst.reshape(N), m16, edge_attr)
    return _final_call(acc, bias.reshape(1, C))


# speedup vs baseline: 1.2965x; 1.2965x over previous
import functools

import jax
import jax.numpy as jnp
from jax import lax
from jax.experimental import pallas as pl
from jax.experimental.pallas import tpu as pltpu
from jax.experimental.pallas import tpu_sc as plsc

N = 10000
E = 320000
D = 128
C = 64
EDGE_DIM = 16

WIDTH = 80
NC, NS, L = 2, 16, 16
NW = NC * NS
ER = E // WIDTH
ER_PT = ER // NW
NR_PT = N // NS
ZR = 25

NBLK = 10
NB = N // NBLK
EBLK = 20
EB = E // EBLK

_NEG = -3e38


def _node_body(x_ref, w_ref, asrc_ref, adst_ref,
               xw_ref, av_ref, bv_ref, ms_ref, md_ref):
    i = pl.program_id(0)
    xw = jnp.dot(x_ref[...], w_ref[...], preferred_element_type=jnp.float32)
    a_s = jnp.sum(xw * asrc_ref[...], axis=1)
    a_d = jnp.sum(xw * adst_ref[...], axis=1)
    ext = (lax.broadcasted_iota(jnp.int32, (NB, WIDTH - C), 1) == 0)
    xw_ref[...] = jnp.concatenate([xw, ext.astype(jnp.float32)], axis=1)
    av_ref[...] = a_s.reshape(1, 1, NB)
    bv_ref[...] = a_d.reshape(1, 1, NB)

    @pl.when(i == 0)
    def _():
        ms_ref[0, 0] = jnp.float32(_NEG)
        md_ref[0, 0] = jnp.float32(_NEG)

    ms_ref[0, 0] = jnp.maximum(ms_ref[0, 0], jnp.max(a_s))
    md_ref[0, 0] = jnp.maximum(md_ref[0, 0], jnp.max(a_d))


_node_call = pl.pallas_call(
    _node_body,
    grid=(NBLK,),
    in_specs=[
        pl.BlockSpec((NB, D), lambda i: (i, 0)),
        pl.BlockSpec((D, C), lambda i: (0, 0)),
        pl.BlockSpec((1, C), lambda i: (0, 0)),
        pl.BlockSpec((1, C), lambda i: (0, 0)),
    ],
    out_specs=[
        pl.BlockSpec((NB, WIDTH), lambda i: (i, 0)),
        pl.BlockSpec((1, 1, NB), lambda i: (i, 0, 0)),
        pl.BlockSpec((1, 1, NB), lambda i: (i, 0, 0)),
        pl.BlockSpec(memory_space=pltpu.SMEM),
        pl.BlockSpec(memory_space=pltpu.SMEM),
    ],
    out_shape=[
        jax.ShapeDtypeStruct((N, WIDTH), jnp.float32),
        jax.ShapeDtypeStruct((NBLK, 1, NB), jnp.float32),
        jax.ShapeDtypeStruct((NBLK, 1, NB), jnp.float32),
        jax.ShapeDtypeStruct((1, 1), jnp.float32),
        jax.ShapeDtypeStruct((1, 1), jnp.float32),
    ],
)


def _edge_body(ea_ref, we_ref, att_ref, ae_ref, me_ref):
    i = pl.program_id(0)
    ve = jnp.sum(we_ref[...] * att_ref[...], axis=1)
    vcol = jnp.concatenate(
        [ve.reshape(EDGE_DIM, 1),
         jnp.zeros((EDGE_DIM, 7), jnp.float32)], axis=1)
    a8 = jnp.dot(ea_ref[...], vcol,
                 preferred_element_type=jnp.float32)
    at = jnp.swapaxes(a8, 0, 1)
    ae_ref[...] = at[0:1].reshape(1, 1, EB)

    @pl.when(i == 0)
    def _():
        me_ref[0, 0] = jnp.float32(_NEG)

    me_ref[0, 0] = jnp.maximum(me_ref[0, 0], jnp.max(at[0:1]))


_edge_call = pl.pallas_call(
    _edge_body,
    grid=(EBLK,),
    in_specs=[
        pl.BlockSpec((EB, EDGE_DIM), lambda i: (i, 0)),
        pl.BlockSpec((EDGE_DIM, C), lambda i: (0, 0)),
        pl.BlockSpec((1, C), lambda i: (0, 0)),
    ],
    out_specs=[
        pl.BlockSpec((1, 1, EB), lambda i: (i, 0, 0)),
        pl.BlockSpec(memory_space=pltpu.SMEM),
    ],
    out_shape=[
        jax.ShapeDtypeStruct((EBLK, 1, EB), jnp.float32),
        jax.ShapeDtypeStruct((1, 1), jnp.float32),
    ],
)


def _sc_body(xw_hbm, src_hbm, dst_hbm, ae_hbm, asrc_hbm, adst_hbm, m_hbm,
             out_hbm,
             acc_sp, vsrc, vdst, vae, vasrc, vadst, vm, gbuf, gbuf2, gbuf3,
             pbuf, zbuf, sem, sem2, sem3, ssem, ssem2, ssem3):
    cid = lax.axis_index("c")
    sid = lax.axis_index("s")
    tid = cid * NS + sid
    nbase = sid * NR_PT
    ebase = tid * ER_PT

    def zrow(r, carry):
        for b in range(WIDTH // L):
            zbuf[r, pl.ds(b * L, L)] = jnp.zeros((L,), jnp.float32)
        return carry

    lax.fori_loop(0, ZR, zrow, 0)

    def zcopy(k, carry):
        pltpu.sync_copy(zbuf, acc_sp.at[pl.ds(nbase + k * ZR, ZR)])
        return carry

    lax.fori_loop(0, NR_PT // ZR, zcopy, 0)

    pltpu.sync_copy(src_hbm.at[pl.ds(ebase, ER_PT)], vsrc)
    pltpu.sync_copy(dst_hbm.at[pl.ds(ebase, ER_PT)], vdst)
    pltpu.sync_copy(ae_hbm.at[pl.ds(ebase, ER_PT)], vae)
    pltpu.sync_copy(asrc_hbm, vasrc)
    pltpu.sync_copy(adst_hbm, vadst)
    pltpu.sync_copy(m_hbm, vm)
    plsc.subcore_barrier()

    mvec = vm[...]

    def compute_p(ci):
        for g in range(WIDTH // L):
            s16 = vsrc[ci, pl.ds(g * L, L)]
            d16 = vdst[ci, pl.ds(g * L, L)]
            a16 = vae[ci, pl.ds(g * L, L)]
            sv = plsc.load_gather(vasrc, [s16])
            dv = plsc.load_gather(vadst, [d16])
            z = sv + dv + a16
            z = jnp.maximum(z, z * jnp.float32(0.2))
            pbuf[pl.ds(g * L, L)] = jnp.exp(z - mvec)

    def scale(ci, gb):
        del ci
        for g in range(WIDTH // L):
            pv = pbuf[pl.ds(g * L, L)]
            for j in range(L):
                e = g * L + j
                pe = pv[j]
                for b in range(WIDTH // L):
                    gb[e, pl.ds(b * L, L)] = gb[e, pl.ds(b * L, L)] * pe

    def wait_gather(gb, s):
        pltpu.make_async_copy(xw_hbm.at[pl.ds(0, WIDTH)], gb, s).wait()

    def wait_scatter(gb, s):
        pltpu.make_async_copy(gb, acc_sp.at[pl.ds(0, WIDTH)], s).wait()

    gbufs = (gbuf, gbuf2, gbuf3)
    gsems = (sem, sem2, sem3)
    ssems = (ssem, ssem2, ssem3)

    def do_chunk(c, k, start_next):
        gb = gbufs[k]
        compute_p(c)
        wait_gather(gb, gsems[k])
        scale(c, gb)
        pltpu.async_copy(gb, acc_sp.at[vdst.at[c]], ssems[k], add=True)
        if start_next:
            kn = (k + 2) % 3
            wait_scatter(gbufs[kn], ssems[kn])
            pltpu.async_copy(xw_hbm.at[vsrc.at[c + 2]], gbufs[kn], gsems[kn])

    pltpu.async_copy(xw_hbm.at[vsrc.at[0]], gbuf, sem)
    pltpu.async_copy(xw_hbm.at[vsrc.at[1]], gbuf2, sem2)

    def tri(c3, carry):
        c = 3 * c3
        for k in range(3):
            do_chunk(c + k, k, True)
        return carry

    do_chunk(0, 0, False)
    pltpu.async_copy(xw_hbm.at[vsrc.at[2]], gbuf3, sem3)

    def tri1(c3, carry):
        c = 3 * c3 + 1
        for k in range(3):
            do_chunk(c + k, (k + 1) % 3, True)
        return carry

    lax.fori_loop(0, (ER_PT - 5) // 3, tri1, 0)
    do_chunk(121, 1, True)
    do_chunk(122, 2, True)
    do_chunk(123, 0, False)
    do_chunk(124, 1, False)
    wait_scatter(gbuf3, ssem3)
    wait_scatter(gbuf, ssem)
    wait_scatter(gbuf2, ssem2)
    plsc.subcore_barrier()

    pltpu.sync_copy(acc_sp.at[pl.ds(nbase, NR_PT)],
                    out_hbm.at[cid, pl.ds(nbase, NR_PT)])


_sc_call = pl.kernel(
    _sc_body,
    out_type=jax.ShapeDtypeStruct((NC, N, WIDTH), jnp.float32),
    mesh=plsc.VectorSubcoreMesh(core_axis_name="c", subcore_axis_name="s"),
    compiler_params=pltpu.CompilerParams(use_tc_tiling_on_sc=False,
                                         needs_layout_passes=False),
    scratch_types=[
        pltpu.VMEM_SHARED((N, WIDTH), jnp.float32),
        pltpu.VMEM((ER_PT, WIDTH), jnp.int32),
        pltpu.VMEM((ER_PT, WIDTH), jnp.int32),
        pltpu.VMEM((ER_PT, WIDTH), jnp.float32),
        pltpu.VMEM((N,), jnp.float32),
        pltpu.VMEM((N,), jnp.float32),
        pltpu.VMEM((L,), jnp.float32),
        pltpu.VMEM((WIDTH, WIDTH), jnp.float32),
        pltpu.VMEM((WIDTH, WIDTH), jnp.float32),
        pltpu.VMEM((WIDTH, WIDTH), jnp.float32),
        pltpu.VMEM((WIDTH,), jnp.float32),
        pltpu.VMEM((ZR, WIDTH), jnp.float32),
        pltpu.SemaphoreType.DMA,
        pltpu.SemaphoreType.DMA,
        pltpu.SemaphoreType.DMA,
        pltpu.SemaphoreType.DMA,
        pltpu.SemaphoreType.DMA,
        pltpu.SemaphoreType.DMA,
    ],
)


def _final_body(acc_ref, bias_ref, out_ref):
    num = acc_ref[0] + acc_ref[1]
    den = num[:, C:C + 1]
    out_ref[...] = num[:, :C] / (den + jnp.float32(1e-16)) + bias_ref[...]


_final_call = pl.pallas_call(
    _final_body,
    grid=(NBLK,),
    in_specs=[
        pl.BlockSpec((NC, NB, WIDTH), lambda i: (0, i, 0)),
        pl.BlockSpec((1, C), lambda i: (0, 0)),
    ],
    out_specs=pl.BlockSpec((NB, C), lambda i: (i, 0)),
    out_shape=jax.ShapeDtypeStruct((N, C), jnp.float32),
)


def kernel(x, edge_index, edge_attr, W, W_e, att_src, att_dst, att_edge, bias):
    src = edge_index[0].astype(jnp.int32)
    dst = edge_index[1].astype(jnp.int32)
    xw_ext, a_src, a_dst, ms, md = _node_call(
        x, W, att_src.reshape(1, C), att_dst.reshape(1, C))
    a_edge, me = _edge_call(edge_attr, W_e, att_edge.reshape(1, C))
    msum = ms[0, 0] + md[0, 0] + me[0, 0]
    m_shift = jnp.maximum(msum, 0.2 * msum)
    m16 = jnp.full((L,), m_shift, jnp.float32)
    acc = _sc_call(xw_ext, src.reshape(ER, WIDTH), dst.reshape(ER, WIDTH),
                   a_edge.reshape(ER, WIDTH), a_src.reshape(N),
                   a_dst.reshape(N), m16)
    return _final_call(acc, bias.reshape(1, C))
